# Initial kernel scaffold; baseline (speedup 1.0000x reference)
import jax
import jax.numpy as jnp
from jax.experimental import pallas as pl


def kernel(x, W):
    raise NotImplementedError("write your pallas kernel here")


# speedup vs baseline: 1.4163x; 1.4163x over previous
import functools

import jax
import jax.numpy as jnp
from jax.experimental import pallas as pl
from jax.experimental.pallas import tpu as pltpu

TOP_K = 8


def _router_kernel(x_ref, w_ref, wout_ref, iout_ref, aux_ref, acc_ref,
                   *, nblocks, n_tokens, num_experts):
    i = pl.program_id(0)
    xb = x_ref[...]
    wt = w_ref[...]
    logits = jax.lax.dot_general(
        xb, wt, dimension_numbers=(((1,), (1,)), ((), ())),
        preferred_element_type=jnp.float32)

    row_max = jnp.max(logits, axis=-1, keepdims=True)
    e = jnp.exp(logits - row_max)
    denom = jnp.sum(e, axis=-1, keepdims=True)
    probs = e / denom

    t = logits.shape[0]
    iota = jax.lax.broadcasted_iota(jnp.int32, (t, num_experts), 1)

    masked = probs
    sel_mask = jnp.zeros((t, num_experts), jnp.float32)
    ws = []
    idxs = []
    for _ in range(TOP_K):
        m = jnp.max(masked, axis=-1, keepdims=True)
        is_max = masked == m
        idx = jnp.min(jnp.where(is_max, iota, num_experts), axis=-1,
                      keepdims=True)
        hit = (iota == idx)
        sel_mask = sel_mask + hit.astype(jnp.float32)
        masked = jnp.where(hit, -1.0, masked)
        ws.append(m)
        idxs.append(idx)

    w_top = jnp.concatenate(ws, axis=-1)
    wout_ref[...] = w_top / jnp.sum(w_top, axis=-1, keepdims=True)
    iout_ref[...] = jnp.concatenate(idxs, axis=-1)

    p_part = jnp.sum(probs, axis=0)
    f_part = jnp.sum(sel_mask, axis=0)

    @pl.when(i == 0)
    def _init():
        acc_ref[...] = jnp.zeros_like(acc_ref)

    acc_ref[0:1, :] += p_part[None, :]
    acc_ref[1:2, :] += f_part[None, :]

    @pl.when(i == nblocks - 1)
    def _finish():
        scale = num_experts / (float(n_tokens) * float(n_tokens))
        aux = scale * jnp.sum(acc_ref[0:1, :] * acc_ref[1:2, :],
                              axis=-1, keepdims=True)
        aux_ref[...] = aux


@jax.jit
def kernel(x, W):
    n, d = x.shape
    num_experts = W.shape[0]
    block_t = 1024 if n % 1024 == 0 else n
    nblocks = n // block_t

    kern = functools.partial(_router_kernel, nblocks=nblocks, n_tokens=n,
                             num_experts=num_experts)
    weights, indices, aux = pl.pallas_call(
        kern,
        grid=(nblocks,),
        in_specs=[
            pl.BlockSpec((block_t, d), lambda i: (i, 0)),
            pl.BlockSpec((num_experts, d), lambda i: (0, 0)),
        ],
        out_specs=[
            pl.BlockSpec((block_t, TOP_K), lambda i: (i, 0)),
            pl.BlockSpec((block_t, TOP_K), lambda i: (i, 0)),
            pl.BlockSpec((1, 1), lambda i: (0, 0)),
        ],
        out_shape=[
            jax.ShapeDtypeStruct((n, TOP_K), jnp.float32),
            jax.ShapeDtypeStruct((n, TOP_K), jnp.int32),
            jax.ShapeDtypeStruct((1, 1), jnp.float32),
        ],
        scratch_shapes=[pltpu.VMEM((2, num_experts), jnp.float32)],
    )(x, W)
    return weights, indices, aux[0, 0]


# speedup vs baseline: 1.6410x; 1.1586x over previous
import functools

import jax
import jax.numpy as jnp
from jax.experimental import pallas as pl
from jax.experimental.pallas import tpu as pltpu

TOP_K = 8


def _router_kernel(x_ref, w_ref, wout_ref, iout_ref, aux_ref, acc_ref,
                   *, nblocks, n_tokens, num_experts):
    i = pl.program_id(0)
    xb = x_ref[...]
    wt = w_ref[...]
    logits = jax.lax.dot_general(
        xb, wt, dimension_numbers=(((1,), (1,)), ((), ())),
        preferred_element_type=jnp.float32)

    row_max = jnp.max(logits, axis=-1, keepdims=True)
    e = jnp.exp(logits - row_max)
    denom = jnp.sum(e, axis=-1, keepdims=True)
    p_part = jnp.sum(e * (1.0 / denom), axis=0)

    t = logits.shape[0]
    iota = jax.lax.broadcasted_iota(jnp.int32, (t, num_experts), 1)
    ebits = jax.lax.bitcast_convert_type(e, jnp.int32)
    key = jax.lax.bitcast_convert_type(
        (ebits & ~(num_experts - 1)) | (num_experts - 1 - iota), jnp.float32)

    sel_mask = jnp.zeros((t, num_experts), jnp.float32)
    ms = []
    for _ in range(TOP_K):
        m = jnp.max(key, axis=-1, keepdims=True)
        hit = key == m
        sel_mask = sel_mask + hit.astype(jnp.float32)
        key = jnp.where(hit, -1.0, key)
        ms.append(m)

    mcat = jax.lax.bitcast_convert_type(
        jnp.concatenate(ms, axis=-1), jnp.int32)
    w_top = jax.lax.bitcast_convert_type(
        mcat & ~(num_experts - 1), jnp.float32)
    wout_ref[...] = w_top / jnp.sum(w_top, axis=-1, keepdims=True)
    iout_ref[...] = (num_experts - 1) - (mcat & (num_experts - 1))

    f_part = jnp.sum(sel_mask, axis=0)

    @pl.when(i == 0)
    def _init():
        acc_ref[...] = jnp.zeros_like(acc_ref)

    acc_ref[0:1, :] += p_part[None, :]
    acc_ref[1:2, :] += f_part[None, :]

    @pl.when(i == nblocks - 1)
    def _finish():
        scale = num_experts / (float(n_tokens) * float(n_tokens))
        aux = scale * jnp.sum(acc_ref[0:1, :] * acc_ref[1:2, :],
                              axis=-1, keepdims=True)
        aux_ref[...] = aux


@jax.jit
def kernel(x, W):
    n, d = x.shape
    num_experts = W.shape[0]
    block_t = 1024 if n % 1024 == 0 else n
    nblocks = n // block_t

    kern = functools.partial(_router_kernel, nblocks=nblocks, n_tokens=n,
                             num_experts=num_experts)
    weights, indices, aux = pl.pallas_call(
        kern,
        grid=(nblocks,),
        in_specs=[
            pl.BlockSpec((block_t, d), lambda i: (i, 0)),
            pl.BlockSpec((num_experts, d), lambda i: (0, 0)),
        ],
        out_specs=[
            pl.BlockSpec((block_t, TOP_K), lambda i: (i, 0)),
            pl.BlockSpec((block_t, TOP_K), lambda i: (i, 0)),
            pl.BlockSpec((1, 1), lambda i: (0, 0)),
        ],
        out_shape=[
            jax.ShapeDtypeStruct((n, TOP_K), jnp.float32),
            jax.ShapeDtypeStruct((n, TOP_K), jnp.int32),
            jax.ShapeDtypeStruct((1, 1), jnp.float32),
        ],
        scratch_shapes=[pltpu.VMEM((2, num_experts), jnp.float32)],
    )(x, W)
    return weights, indices, aux[0, 0]
